# speedup vs baseline: 1.7935x; 1.7935x over previous
import functools

import jax
import jax.numpy as jnp
from jax import lax
from jax.experimental import pallas as pl
from jax.experimental.pallas import tpu as pltpu
from jax.experimental.pallas import tpu_sc as plsc

SEM_SZ = 4096 + 2
ACU_SZ = 2048 + 2
N_CB = 9
SEQ = 2048

NC, NS, L = 2, 16, 16
NW = NC * NS

CHUNK = 32
NBUF = 3
AHEAD = NBUF - 1


@functools.lru_cache(maxsize=None)
def _build(b_tot, d):
    b_per_w = b_tot // NW
    n_chunks = b_per_w // CHUNK
    n_groups = b_per_w // L
    mesh = plsc.VectorSubcoreMesh(core_axis_name="c", subcore_axis_name="s")

    @functools.partial(
        pl.kernel,
        out_type=jax.ShapeDtypeStruct((b_tot, d), jnp.float32),
        mesh=mesh,
        scratch_types=[
            pltpu.VMEM((b_per_w,), jnp.int32),
            pltpu.VMEM((NBUF, CHUNK, d), jnp.float32),
            pltpu.SemaphoreType.DMA,
            pltpu.SemaphoreType.DMA,
        ],
    )
    def gather_kernel(ids_hbm, table_hbm, out_hbm, idx_v, rows_v, gsem, ssem):
        wid = lax.axis_index("s") * NC + lax.axis_index("c")
        base = wid * b_per_w

        pltpu.sync_copy(ids_hbm.at[pl.ds(base, b_per_w)], idx_v)

        def fix(g, carry):
            pos = base + g * L
            cb = (pos // SEQ) % N_CB
            off = jnp.where(cb == 0, 0, (SEM_SZ - ACU_SZ) + ACU_SZ * cb)
            idx_v[pl.ds(g * L, L)] = idx_v[pl.ds(g * L, L)] + off
            return carry

        lax.fori_loop(0, n_groups, fix, 0)

        def g_copy(k, b):
            return pltpu.make_async_copy(
                table_hbm.at[idx_v.at[pl.ds(k * CHUNK, CHUNK)]],
                rows_v.at[b],
                gsem,
            )

        def s_copy(k, b):
            return pltpu.make_async_copy(
                rows_v.at[b],
                out_hbm.at[pl.ds(base + k * CHUNK, CHUNK)],
                ssem,
            )

        for b in range(AHEAD):
            g_copy(b, b).start()

        def outer(g, carry):
            for b in range(NBUF):
                k = g * NBUF + b
                nk = k + AHEAD
                nb = (b + AHEAD) % NBUF

                @pl.when(nk < n_chunks)
                def _():
                    @pl.when(k >= 1)
                    def _():
                        s_copy(0, nb).wait()

                    g_copy(nk, nb).start()

                g_copy(k, b).wait()
                s_copy(k, b).start()
            return carry

        lax.fori_loop(0, n_chunks // NBUF, outer, 0)

        for j in range(AHEAD + 1):
            kk = n_chunks - 1 - j
            s_copy(kk, kk % NBUF).wait()

    return gather_kernel


def kernel(input_ids, emb_table):
    bsz, ncb, seq = input_ids.shape
    v, d = emb_table.shape
    ids = input_ids.reshape(-1).astype(jnp.int32)
    out = _build(bsz * ncb * seq, d)(ids, emb_table)
    return out.reshape(bsz, ncb, seq, d)


# speedup vs baseline: 1.7983x; 1.0027x over previous
import functools

import jax
import jax.numpy as jnp
from jax import lax
from jax.experimental import pallas as pl
from jax.experimental.pallas import tpu as pltpu
from jax.experimental.pallas import tpu_sc as plsc

SEM_SZ = 4096 + 2
ACU_SZ = 2048 + 2
N_CB = 9
SEQ = 2048

NC, NS, L = 2, 16, 16
NW = NC * NS

CHUNK = 16
NBUF = 6
AHEAD = 4


@functools.lru_cache(maxsize=None)
def _build(b_tot, d):
    b_per_w = b_tot // NW
    n_chunks = b_per_w // CHUNK
    n_groups = b_per_w // L
    mesh = plsc.VectorSubcoreMesh(core_axis_name="c", subcore_axis_name="s")

    @functools.partial(
        pl.kernel,
        out_type=jax.ShapeDtypeStruct((b_tot, d), jnp.float32),
        mesh=mesh,
        scratch_types=[
            pltpu.VMEM((b_per_w,), jnp.int32),
            pltpu.VMEM((NBUF, CHUNK, d), jnp.float32),
            pltpu.SemaphoreType.DMA,
            pltpu.SemaphoreType.DMA,
        ],
    )
    def gather_kernel(ids_hbm, table_hbm, out_hbm, idx_v, rows_v, gsem, ssem):
        wid = lax.axis_index("s") * NC + lax.axis_index("c")
        base = wid * b_per_w

        pltpu.sync_copy(ids_hbm.at[pl.ds(base, b_per_w)], idx_v)

        def fix(g, carry):
            pos = base + g * L
            cb = (pos // SEQ) % N_CB
            off = jnp.where(cb == 0, 0, (SEM_SZ - ACU_SZ) + ACU_SZ * cb)
            idx_v[pl.ds(g * L, L)] = idx_v[pl.ds(g * L, L)] + off
            return carry

        lax.fori_loop(0, n_groups, fix, 0)

        def g_copy(k, b):
            return pltpu.make_async_copy(
                table_hbm.at[idx_v.at[pl.ds(k * CHUNK, CHUNK)]],
                rows_v.at[b],
                gsem,
            )

        def s_copy(k, b):
            return pltpu.make_async_copy(
                rows_v.at[b],
                out_hbm.at[pl.ds(base + k * CHUNK, CHUNK)],
                ssem,
            )

        for b in range(AHEAD):
            g_copy(b, b).start()

        def outer(g, carry):
            for b in range(NBUF):
                k = g * NBUF + b
                nk = k + AHEAD
                nb = (b + AHEAD) % NBUF

                @pl.when(nk < n_chunks)
                def _():
                    @pl.when(k >= NBUF - AHEAD)
                    def _():
                        s_copy(0, nb).wait()

                    g_copy(nk, nb).start()

                g_copy(k, b).wait()
                s_copy(k, b).start()
            return carry

        lax.fori_loop(0, n_chunks // NBUF, outer, 0)

        for j in range(NBUF):
            kk = n_chunks - 1 - j
            s_copy(kk, kk % NBUF).wait()

    return gather_kernel


def kernel(input_ids, emb_table):
    bsz, ncb, seq = input_ids.shape
    v, d = emb_table.shape
    ids = input_ids.reshape(-1).astype(jnp.int32)
    out = _build(bsz * ncb * seq, d)(ids, emb_table)
    return out.reshape(bsz, ncb, seq, d)
